# baseline (speedup 1.0000x reference)
import functools

import jax
import jax.numpy as jnp
from jax import lax
from jax.experimental import pallas as pl
from jax.experimental.pallas import tpu as pltpu
from jax.experimental.pallas import tpu_sc as plsc

B = 32
P = 16
PS = 32
D = 64
SR = 2048
SL = 1024
CH = 2 * D + 1

NC = 2
NS = 16
NW = NC * NS
TASKS = B * P
TPW = TASKS // NW


def _sc_body(seq1_hbm, seq2_hbm, ridx_hbm, cidx_hbm, geo_hbm, out_hbm,
             ridx_v, cidx_v, row_v, col_v, geo_v, sem_in, sem_out):
    c = lax.axis_index("c")
    s = lax.axis_index("s")
    wid = s * NC + c

    def task_body(t, _):
        task = wid * TPW + t
        i = task // P
        p = lax.rem(task, P)
        base = pl.multiple_of(task * PS, PS)
        pltpu.sync_copy(ridx_hbm.at[pl.ds(base, PS)], ridx_v)
        pltpu.sync_copy(cidx_hbm.at[pl.ds(base, PS)], cidx_v)
        pltpu.async_copy(seq1_hbm.at[ridx_v], row_v, sem_in).wait()
        pltpu.async_copy(seq2_hbm.at[cidx_v], col_v, sem_in).wait()
        pltpu.sync_copy(geo_hbm.at[i, p], geo_v)
        copies = []
        for a in range(PS):
            copies.append(pltpu.async_copy(
                row_v, out_hbm.at[i, p, a, :, pl.ds(0, D)], sem_out))
        for b in range(PS):
            copies.append(pltpu.async_copy(
                col_v, out_hbm.at[i, p, :, b, pl.ds(D, D)], sem_out))
        for a in range(PS):
            copies.append(pltpu.async_copy(
                geo_v.at[a], out_hbm.at[i, p, a, :, pl.ds(2 * D, 1)], sem_out))
        for cp in copies:
            cp.wait()
        return ()

    lax.fori_loop(0, TPW, task_body, (), unroll=False)


def kernel(seq1M, seq2M, patches, geo):
    seq1f = seq1M.reshape(B * SR, D)
    seq2f = seq2M.reshape(B * SL, D)
    off = jnp.arange(PS, dtype=jnp.int32)
    bb = jnp.arange(B, dtype=jnp.int32)[:, None, None]
    ridx = (bb * SR + patches[:, :, 0][:, :, None] + off).reshape(-1)
    cidx = (bb * SL + patches[:, :, 1][:, :, None] + off).reshape(-1)
    geo3 = geo[..., None]

    run = pl.kernel(
        _sc_body,
        out_type=jax.ShapeDtypeStruct((B, P, PS, PS, CH), jnp.float32),
        mesh=plsc.VectorSubcoreMesh(core_axis_name="c", subcore_axis_name="s"),
        compiler_params=pltpu.CompilerParams(use_tc_tiling_on_sc=False),
        scratch_types=[
            pltpu.VMEM((PS,), jnp.int32),
            pltpu.VMEM((PS,), jnp.int32),
            pltpu.VMEM((PS, D), jnp.float32),
            pltpu.VMEM((PS, D), jnp.float32),
            pltpu.VMEM((PS, PS, 1), jnp.float32),
            pltpu.SemaphoreType.DMA,
            pltpu.SemaphoreType.DMA,
        ],
    )
    return run(seq1f, seq2f, ridx, cidx, geo3)


# speedup vs baseline: 3.1449x; 3.1449x over previous
import jax
import jax.numpy as jnp
from jax import lax
from jax.experimental import pallas as pl
from jax.experimental.pallas import tpu as pltpu
from jax.experimental.pallas import tpu_sc as plsc

B = 32
P = 16
PS = 32
D = 64
SR = 2048
SL = 1024
CH = 2 * D + 1
L = 16

NC = 2
NS = 16
NW = NC * NS
TASKS = B * P
TPW = TASKS // NW

AC = 2
NBUF = 4
GROUPS = PS // (AC * NBUF)
SPAIRS = 32


def _sc_body(seq1_hbm, seq2_hbm, desc_hbm, geo_hbm, out_hbm,
             pv, row_v, col_v, geo_v,
             buf0, buf1, buf2, buf3,
             sem_in, sem0, sem1, sem2, sem3):
    c = lax.axis_index("c")
    s = lax.axis_index("s")
    wid = s * NC + c
    bufs = (buf0, buf1, buf2, buf3)
    sems = (sem0, sem1, sem2, sem3)
    iota = lax.iota(jnp.int32, L)

    def task_body(t, _):
        task = wid * TPW + t
        i = task // P
        p = lax.rem(task, P)
        pltpu.sync_copy(desc_hbm.at[task], pv)
        pvec = pv[pl.ds(0, L)]
        q8r = pl.multiple_of(pvec[0], 8)
        ro = pvec[1]
        q8c = pl.multiple_of(pvec[2], 8)
        co = pvec[3]
        pltpu.async_copy(seq1_hbm.at[i, pl.ds(q8r, SPAIRS), :], row_v, sem_in)
        pltpu.async_copy(seq2_hbm.at[i, pl.ds(q8c, SPAIRS), :], col_v, sem_in)
        pltpu.sync_copy(geo_hbm.at[i, p], geo_v)
        pltpu.make_async_copy(
            seq1_hbm.at[i, pl.ds(0, SPAIRS), :], row_v, sem_in).wait()
        pltpu.make_async_copy(
            seq2_hbm.at[i, pl.ds(0, SPAIRS), :], col_v, sem_in).wait()

        def fill_group(g, _):
            for k in range(NBUF):
                buf = bufs[k]
                a0 = (g * NBUF + k) * AC

                @pl.when(jnp.logical_not((t == 0) & (g == 0)))
                def _reclaim(buf=buf, sem=sems[k]):
                    pltpu.make_async_copy(
                        out_hbm.at[0, 0, pl.ds(0, AC)], buf, sem).wait()
                for b in range(PS):
                    pr = (ro + b) // 2
                    hf = ((ro + b) % 2) * D
                    for j in range(D // L):
                        xr = row_v[pr, pl.ds(hf + j * L, L)]
                        for m in range(AC):
                            buf[m, b, pl.ds(j * L, L)] = xr
                for m in range(AC):
                    a = a0 + m
                    pc = (co + a) // 2
                    hc = ((co + a) % 2) * D
                    for j in range(D // L):
                        xc = col_v[pc, pl.ds(hc + j * L, L)]
                        for b in range(PS):
                            buf[m, b, pl.ds(D + j * L, L)] = xc
                    gs = a // 4
                    go = (a % 4) * PS
                    for h in range(PS // L):
                        xg = geo_v[gs, pl.ds(go + h * L, L)]
                        plsc.store_scatter(
                            buf,
                            [jnp.full((L,), m, jnp.int32),
                             iota + (h * L),
                             jnp.full((L,), CH - 1, jnp.int32)],
                            xg)
                pltpu.async_copy(buf, out_hbm.at[i, p, pl.ds(a0, AC)], sems[k])
            return ()

        lax.fori_loop(0, GROUPS, fill_group, (), unroll=False)
        return ()

    lax.fori_loop(0, TPW, task_body, (), unroll=False)

    for k in range(NBUF):
        pltpu.make_async_copy(
            out_hbm.at[0, 0, pl.ds(0, AC)], bufs[k], sems[k]).wait()


def kernel(seq1M, seq2M, patches, geo):
    seq1p = seq1M.reshape(B, SR // 2, 2 * D)
    seq2p = seq2M.reshape(B, SL // 2, 2 * D)
    geo8 = geo.reshape(B, P, PS * PS // 128, 128)
    r = patches[:, :, 0].reshape(TASKS).astype(jnp.int32)
    cc = patches[:, :, 1].reshape(TASKS).astype(jnp.int32)
    q8r = jnp.minimum((r // 16) * 8, SR // 2 - SPAIRS)
    q8c = jnp.minimum((cc // 16) * 8, SL // 2 - SPAIRS)
    desc = jnp.stack([q8r, r - 2 * q8r, q8c, cc - 2 * q8c], axis=1)
    desc = jnp.pad(desc, ((0, 0), (0, 128 - 4)))

    run = pl.kernel(
        _sc_body,
        out_type=jax.ShapeDtypeStruct((B, P, PS, PS, CH), jnp.float32),
        mesh=plsc.VectorSubcoreMesh(core_axis_name="c", subcore_axis_name="s"),
        compiler_params=pltpu.CompilerParams(needs_layout_passes=False),
        scratch_types=[
            pltpu.VMEM((128,), jnp.int32),
            pltpu.VMEM((SPAIRS, 2 * D), jnp.float32),
            pltpu.VMEM((SPAIRS, 2 * D), jnp.float32),
            pltpu.VMEM((PS * PS // 128, 128), jnp.float32),
            pltpu.VMEM((AC, PS, CH), jnp.float32),
            pltpu.VMEM((AC, PS, CH), jnp.float32),
            pltpu.VMEM((AC, PS, CH), jnp.float32),
            pltpu.VMEM((AC, PS, CH), jnp.float32),
            pltpu.SemaphoreType.DMA,
            pltpu.SemaphoreType.DMA,
            pltpu.SemaphoreType.DMA,
            pltpu.SemaphoreType.DMA,
            pltpu.SemaphoreType.DMA,
        ],
    )
    return run(seq1p, seq2p, desc, geo8)


# speedup vs baseline: 3.6115x; 1.1484x over previous
import jax
import jax.numpy as jnp
from jax import lax
from jax.experimental import pallas as pl
from jax.experimental.pallas import tpu as pltpu
from jax.experimental.pallas import tpu_sc as plsc

B = 32
P = 16
PS = 32
D = 64
SR = 2048
SL = 1024
CH = 2 * D + 1
L = 16

NC = 2
NS = 16
NW = NC * NS
TASKS = B * P
TPW = TASKS // NW

AC = 4
NBUF = 2
GROUPS = PS // (AC * NBUF)
SPAIRS = 32


def _sc_body(seq1_hbm, seq2_hbm, desc_hbm, geo_hbm, out_hbm,
             pv, row_v, col_v, geo_v,
             buf0, buf1,
             sem_in, sem0, sem1):
    c = lax.axis_index("c")
    s = lax.axis_index("s")
    wid = s * NC + c
    bufs = (buf0, buf1)
    sems = (sem0, sem1)
    iota = lax.iota(jnp.int32, L)

    def task_body(t, _):
        task = wid * TPW + t
        i = task // P
        p = lax.rem(task, P)
        pltpu.sync_copy(desc_hbm.at[task], pv)
        pvec = pv[pl.ds(0, L)]
        q8r = pl.multiple_of(pvec[0], 8)
        ro = pvec[1]
        q8c = pl.multiple_of(pvec[2], 8)
        co = pvec[3]
        pltpu.async_copy(seq1_hbm.at[i, pl.ds(q8r, SPAIRS), :], row_v, sem_in)
        pltpu.async_copy(seq2_hbm.at[i, pl.ds(q8c, SPAIRS), :], col_v, sem_in)
        pltpu.sync_copy(geo_hbm.at[i, p], geo_v)
        pltpu.make_async_copy(
            seq1_hbm.at[i, pl.ds(0, SPAIRS), :], row_v, sem_in).wait()
        pltpu.make_async_copy(
            seq2_hbm.at[i, pl.ds(0, SPAIRS), :], col_v, sem_in).wait()

        def fill_group(g, _):
            for k in range(NBUF):
                buf = bufs[k]
                a0 = (g * NBUF + k) * AC

                @pl.when(jnp.logical_not((t == 0) & (g == 0)))
                def _reclaim(buf=buf, sem=sems[k]):
                    pltpu.make_async_copy(
                        out_hbm.at[0, 0, pl.ds(0, AC)], buf, sem).wait()
                for b in range(PS):
                    pr = (ro + b) // 2
                    hf = ((ro + b) % 2) * D
                    for j in range(D // L):
                        xr = row_v[pr, pl.ds(hf + j * L, L)]
                        for m in range(AC):
                            buf[m, b, pl.ds(j * L, L)] = xr
                for m in range(AC):
                    a = a0 + m
                    pc = (co + a) // 2
                    hc = ((co + a) % 2) * D
                    for j in range(D // L):
                        xc = col_v[pc, pl.ds(hc + j * L, L)]
                        for b in range(PS):
                            buf[m, b, pl.ds(D + j * L, L)] = xc
                    gs = a // 4
                    go = (a % 4) * PS
                    for h in range(PS // L):
                        xg = geo_v[gs, pl.ds(go + h * L, L)]
                        plsc.store_scatter(
                            buf,
                            [jnp.full((L,), m, jnp.int32),
                             iota + (h * L),
                             jnp.full((L,), CH - 1, jnp.int32)],
                            xg)
                pltpu.async_copy(buf, out_hbm.at[i, p, pl.ds(a0, AC)], sems[k])
            return ()

        lax.fori_loop(0, GROUPS, fill_group, (), unroll=False)
        return ()

    lax.fori_loop(0, TPW, task_body, (), unroll=False)

    for k in range(NBUF):
        pltpu.make_async_copy(
            out_hbm.at[0, 0, pl.ds(0, AC)], bufs[k], sems[k]).wait()


def kernel(seq1M, seq2M, patches, geo):
    seq1p = seq1M.reshape(B, SR // 2, 2 * D)
    seq2p = seq2M.reshape(B, SL // 2, 2 * D)
    geo8 = geo.reshape(B, P, PS * PS // 128, 128)
    r = patches[:, :, 0].reshape(TASKS).astype(jnp.int32)
    cc = patches[:, :, 1].reshape(TASKS).astype(jnp.int32)
    q8r = jnp.minimum((r // 16) * 8, SR // 2 - SPAIRS)
    q8c = jnp.minimum((cc // 16) * 8, SL // 2 - SPAIRS)
    desc = jnp.stack([q8r, r - 2 * q8r, q8c, cc - 2 * q8c], axis=1)
    desc = jnp.pad(desc, ((0, 0), (0, 128 - 4)))

    run = pl.kernel(
        _sc_body,
        out_type=jax.ShapeDtypeStruct((B, P, PS, PS, CH), jnp.float32),
        mesh=plsc.VectorSubcoreMesh(core_axis_name="c", subcore_axis_name="s"),
        compiler_params=pltpu.CompilerParams(needs_layout_passes=False),
        scratch_types=[
            pltpu.VMEM((128,), jnp.int32),
            pltpu.VMEM((SPAIRS, 2 * D), jnp.float32),
            pltpu.VMEM((SPAIRS, 2 * D), jnp.float32),
            pltpu.VMEM((PS * PS // 128, 128), jnp.float32),
            pltpu.VMEM((AC, PS, CH), jnp.float32),
            pltpu.VMEM((AC, PS, CH), jnp.float32),
            pltpu.SemaphoreType.DMA,
            pltpu.SemaphoreType.DMA,
            pltpu.SemaphoreType.DMA,
        ],
    )
    return run(seq1p, seq2p, desc, geo8)
